# baseline (speedup 1.0000x reference)
import functools

import jax
import jax.numpy as jnp
from jax.experimental import pallas as pl
from jax.experimental.pallas import tpu as pltpu


def _edge_body(g_ref, ee_ref, w_ref, b_ref, sig_ref, eenew_ref):
    x = jnp.dot(ee_ref[...], w_ref[...], preferred_element_type=jnp.float32)
    x = x + b_ref[...] + g_ref[...]
    sig_ref[...] = jax.nn.sigmoid(x)
    eenew_ref[...] = ee_ref[...] + jnp.maximum(x, 0.0)


def _edge_stage(g, ee, w4, b4, block_e=2000):
    E, H = ee.shape
    grid = (E // block_e,)
    b4 = b4.reshape(1, H)
    return pl.pallas_call(
        _edge_body,
        grid=grid,
        in_specs=[
            pl.BlockSpec((block_e, H), lambda i: (i, 0)),
            pl.BlockSpec((block_e, H), lambda i: (i, 0)),
            pl.BlockSpec((H, H), lambda i: (0, 0)),
            pl.BlockSpec((1, H), lambda i: (0, 0)),
        ],
        out_specs=[
            pl.BlockSpec((block_e, H), lambda i: (i, 0)),
            pl.BlockSpec((block_e, H), lambda i: (i, 0)),
        ],
        out_shape=[
            jax.ShapeDtypeStruct((E, H), jnp.float32),
            jax.ShapeDtypeStruct((E, H), jnp.float32),
        ],
    )(g, ee, w4, b4)


def kernel(h, edge_index, e, emb_h, We, be, layerW, layerB, W0, b0, W1, b1, W2, b2):
    N = h.shape[0]
    L = layerW.shape[0]
    src = edge_index[0]
    dst = edge_index[1]
    hh = emb_h[h]
    ee = e @ We + be
    for l in range(L):
        W = layerW[l]
        B = layerB[l]
        h_in, e_in = hh, ee
        Ah = hh @ W[0] + B[0]
        Bh = hh @ W[1] + B[1]
        Dh = hh @ W[2] + B[2]
        Eh = hh @ W[3] + B[3]
        g = Dh[src] + Eh[dst]
        sigma, ee = _edge_stage(g, e_in, W[4], B[4])
        num = jax.ops.segment_sum(sigma * Bh[src], dst, num_segments=N)
        den = jax.ops.segment_sum(sigma, dst, num_segments=N)
        hh = h_in + jnp.maximum(Ah + num / (den + 1e-6), 0.0)
    y = jnp.maximum(hh @ W0 + b0, 0.0)
    y = jnp.maximum(y @ W1 + b1, 0.0)
    y = y @ W2 + b2
    return y


# speedup vs baseline: 3.2440x; 3.2440x over previous
import functools

import jax
import jax.numpy as jnp
from jax import lax
from jax.experimental import pallas as pl
from jax.experimental.pallas import tpu as pltpu
from jax.experimental.pallas import tpu_sc as plsc

_N = 10000
_E = 320000
_H = 128
_NSC = 2
_NS = 16
_IR = 80
_BC = 400
_RPC = _BC // _IR
_BCS = 160
_RPCS = _BCS // _IR
_EPT = _E // _NS
_NCH = _EPT // _BCS
_EPTG = _E // (_NSC * _NS)
_NCHG = _EPTG // _BC
_AR0 = 624
_ARN = 640

_MESH = plsc.VectorSubcoreMesh(
    core_axis_name="c", subcore_axis_name="s", num_cores=_NSC, num_subcores=_NS)


@functools.partial(
    pl.kernel,
    out_type=jax.ShapeDtypeStruct((_E, _H), jnp.float32),
    mesh=_MESH,
    scratch_types=[
        pltpu.VMEM((_BC, _H), jnp.float32),
        pltpu.VMEM((_BC, _H), jnp.float32),
        pltpu.VMEM((_RPC, _IR), jnp.int32),
        pltpu.VMEM((_RPC, _IR), jnp.int32),
        pltpu.SemaphoreType.DMA,
    ],
)
def _sc_gather(dh, eh, src_i, dst_i, g, buf_d, buf_e, src_b, dst_b, sem):
    c = lax.axis_index("c")
    s = lax.axis_index("s")

    def chunk(j, carry):
        base = (c * _NS + s) * _EPTG + j * _BC
        pltpu.sync_copy(src_i.at[c, s, j], src_b)
        pltpu.sync_copy(dst_i.at[c, s, j], dst_b)
        descs = []
        for q in range(_RPC):
            descs.append(pltpu.async_copy(
                dh.at[src_b.at[q]], buf_d.at[pl.ds(q * _IR, _IR)], sem))
            descs.append(pltpu.async_copy(
                eh.at[dst_b.at[q]], buf_e.at[pl.ds(q * _IR, _IR)], sem))
        for d in descs:
            d.wait()

        def addrow(r, carry2):
            for v in range(_H // 16):
                sl = pl.ds(v * 16, 16)
                buf_d[r, sl] = buf_d[r, sl] + buf_e[r, sl]
            return carry2

        lax.fori_loop(0, _BC, addrow, 0)
        pltpu.sync_copy(buf_d, g.at[pl.ds(base, _BC)])
        return carry

    lax.fori_loop(0, _NCHG, chunk, 0)


@functools.partial(
    pl.kernel,
    out_type=jax.ShapeDtypeStruct((_NSC, _N, _H), jnp.float32),
    mesh=_MESH,
    scratch_types=[
        pltpu.VMEM((_BCS, _H), jnp.float32),
        pltpu.VMEM((_BCS, _H), jnp.float32),
        pltpu.VMEM((_RPCS, _IR), jnp.int32),
        pltpu.VMEM((_RPCS, _IR), jnp.int32),
        pltpu.VMEM_SHARED((_N, _H), jnp.float32),
        pltpu.SemaphoreType.DMA,
        pltpu.SemaphoreType.DMA,
    ],
)
def _sc_scatter(sig, bh, src_i, dst_i, acc_out,
                sig_b, b_b, src_b, dst_b, acc, gsem, ssem):
    c = lax.axis_index("c")
    s = lax.axis_index("s")

    zeros = jnp.zeros((16,), jnp.float32)

    def zrow(r, carry):
        for v in range(_H // 16):
            b_b[r, pl.ds(v * 16, 16)] = zeros
        return carry

    lax.fori_loop(0, _BCS, zrow, 0)
    r0 = s * _AR0
    for t in range(_ARN // _BCS):
        pltpu.sync_copy(b_b, acc.at[pl.ds(r0 + t * _BCS, _BCS)])
    plsc.subcore_barrier()

    def chunk(j, carry):
        base = s * _EPT + j * _BCS
        pltpu.sync_copy(dst_i.at[s, j], dst_b)

        @pl.when(c == 0)
        def _num():
            pltpu.sync_copy(src_i.at[s, j], src_b)
            descs = []
            for q in range(_RPCS):
                descs.append(pltpu.async_copy(
                    bh.at[src_b.at[q]], b_b.at[pl.ds(q * _IR, _IR)], gsem))
            pltpu.sync_copy(sig.at[pl.ds(base, _BCS)], sig_b)
            for d in descs:
                d.wait()

            def mulrow(r, carry2):
                for v in range(_H // 16):
                    sl = pl.ds(v * 16, 16)
                    b_b[r, sl] = sig_b[r, sl] * b_b[r, sl]
                return carry2

            lax.fori_loop(0, _BCS, mulrow, 0)
            sdescs = []
            for q in range(_RPCS):
                sdescs.append(pltpu.async_copy(
                    b_b.at[pl.ds(q * _IR, _IR)], acc.at[dst_b.at[q]], ssem,
                    add=True))
            for d in sdescs:
                d.wait()

        @pl.when(c == 1)
        def _den():
            pltpu.sync_copy(sig.at[pl.ds(base, _BCS)], sig_b)
            sdescs = []
            for q in range(_RPCS):
                sdescs.append(pltpu.async_copy(
                    sig_b.at[pl.ds(q * _IR, _IR)], acc.at[dst_b.at[q]], ssem,
                    add=True))
            for d in sdescs:
                d.wait()

        return carry

    lax.fori_loop(0, _NCH, chunk, 0)
    plsc.subcore_barrier()
    pltpu.sync_copy(acc.at[pl.ds(r0, _ARN)], acc_out.at[c, pl.ds(r0, _ARN)])


_BN = 2000
_BE = 2000


def _node_matmuls(hh, w_ref, b_ref, hh_ref, ah_ref, bh_ref, dh_ref, eh_ref):
    hh_ref[...] = hh
    ah_ref[...] = jnp.dot(hh, w_ref[0], preferred_element_type=jnp.float32) + b_ref[0]
    bh_ref[...] = jnp.dot(hh, w_ref[1], preferred_element_type=jnp.float32) + b_ref[1]
    dh_ref[...] = jnp.dot(hh, w_ref[2], preferred_element_type=jnp.float32) + b_ref[2]
    eh_ref[...] = jnp.dot(hh, w_ref[3], preferred_element_type=jnp.float32) + b_ref[3]


def _node0_body(h_ref, emb_ref, w_ref, b_ref, *out_refs):
    onehot = (h_ref[...] == lax.broadcasted_iota(jnp.int32, (_BN, _H), 1)
              ).astype(jnp.float32)
    hh = jnp.dot(onehot, emb_ref[...], preferred_element_type=jnp.float32)
    _node_matmuls(hh, w_ref, b_ref, *out_refs)


def _hupdate(hprev_ref, ah_ref, acc_ref):
    return hprev_ref[...] + jnp.maximum(
        ah_ref[...] + acc_ref[0] / (acc_ref[1] + 1e-6), 0.0)


def _node_body(hprev_ref, ahprev_ref, acc_ref, w_ref, b_ref, *out_refs):
    hh = _hupdate(hprev_ref, ahprev_ref, acc_ref)
    _node_matmuls(hh, w_ref, b_ref, *out_refs)


def _node_specs():
    ins = [
        pl.BlockSpec((5, _H, _H), lambda i: (0, 0, 0)),
        pl.BlockSpec((5, 1, _H), lambda i: (0, 0, 0)),
    ]
    outs = [pl.BlockSpec((_BN, _H), lambda i: (i, 0))] * 5
    out_shape = [jax.ShapeDtypeStruct((_N, _H), jnp.float32)] * 5
    return ins, outs, out_shape


def _tc_node0(h2, emb, w, b):
    ins, outs, out_shape = _node_specs()
    return pl.pallas_call(
        _node0_body,
        grid=(_N // _BN,),
        in_specs=[pl.BlockSpec((_BN, 1), lambda i: (i, 0)),
                  pl.BlockSpec((_H, _H), lambda i: (0, 0))] + ins,
        out_specs=outs,
        out_shape=out_shape,
    )(h2, emb, w, b)


def _tc_node(hprev, ahprev, acc, w, b):
    ins, outs, out_shape = _node_specs()
    return pl.pallas_call(
        _node_body,
        grid=(_N // _BN,),
        in_specs=[pl.BlockSpec((_BN, _H), lambda i: (i, 0)),
                  pl.BlockSpec((_BN, _H), lambda i: (i, 0)),
                  pl.BlockSpec((2, _BN, _H), lambda i: (0, i, 0))] + ins,
        out_specs=outs,
        out_shape=out_shape,
    )(hprev, ahprev, acc, w, b)


def _edge_math(ee, g_ref, w_ref, b_ref, sig_ref, eenew_ref):
    x = jnp.dot(ee, w_ref[...], preferred_element_type=jnp.float32)
    x = x + b_ref[...] + g_ref[...]
    sig_ref[...] = jax.nn.sigmoid(x)
    if eenew_ref is not None:
        eenew_ref[...] = ee + jnp.maximum(x, 0.0)


def _edge0_body(e_ref, we_ref, be_ref, g_ref, w_ref, b_ref, sig_ref, eenew_ref):
    ee = e_ref[...] * we_ref[...] + be_ref[...]
    _edge_math(ee, g_ref, w_ref, b_ref, sig_ref, eenew_ref)


def _edge_body(ee_ref, g_ref, w_ref, b_ref, sig_ref, eenew_ref=None):
    _edge_math(ee_ref[...], g_ref, w_ref, b_ref, sig_ref, eenew_ref)


_EDGE_SPEC = pl.BlockSpec((_BE, _H), lambda i: (i, 0))
_EDGE_SHAPE = jax.ShapeDtypeStruct((_E, _H), jnp.float32)


def _edge_wspecs():
    return [_EDGE_SPEC,
            pl.BlockSpec((_H, _H), lambda i: (0, 0)),
            pl.BlockSpec((1, _H), lambda i: (0, 0))]


def _tc_edge0(e, we, be, g, w4, b4):
    return pl.pallas_call(
        _edge0_body,
        grid=(_E // _BE,),
        in_specs=[pl.BlockSpec((_BE, 1), lambda i: (i, 0)),
                  pl.BlockSpec((1, _H), lambda i: (0, 0)),
                  pl.BlockSpec((1, _H), lambda i: (0, 0))] + _edge_wspecs(),
        out_specs=[_EDGE_SPEC, _EDGE_SPEC],
        out_shape=[_EDGE_SHAPE, _EDGE_SHAPE],
    )(e, we, be, g, w4, b4)


def _tc_edge(ee, g, w4, b4, want_ee):
    out_specs = [_EDGE_SPEC, _EDGE_SPEC] if want_ee else [_EDGE_SPEC]
    out_shape = [_EDGE_SHAPE, _EDGE_SHAPE] if want_ee else [_EDGE_SHAPE]
    return pl.pallas_call(
        _edge_body,
        grid=(_E // _BE,),
        in_specs=[_EDGE_SPEC] + _edge_wspecs(),
        out_specs=out_specs,
        out_shape=out_shape,
    )(ee, g, w4, b4)


def _readout_body(hprev_ref, ahprev_ref, acc_ref, w0, b0, w1, b1, w2, b2, y_ref):
    hh = _hupdate(hprev_ref, ahprev_ref, acc_ref)
    y = jnp.maximum(jnp.dot(hh, w0[...], preferred_element_type=jnp.float32)
                    + b0[...], 0.0)
    y = jnp.maximum(jnp.dot(y, w1[...], preferred_element_type=jnp.float32)
                    + b1[...], 0.0)
    y_ref[...] = jnp.dot(y, w2[...], preferred_element_type=jnp.float32) + b2[...]


def _tc_readout(hprev, ahprev, acc, W0, b0, W1, b1, W2, b2):
    H2, H4, NC = W0.shape[1], W1.shape[1], W2.shape[1]
    return pl.pallas_call(
        _readout_body,
        grid=(_N // _BN,),
        in_specs=[pl.BlockSpec((_BN, _H), lambda i: (i, 0)),
                  pl.BlockSpec((_BN, _H), lambda i: (i, 0)),
                  pl.BlockSpec((2, _BN, _H), lambda i: (0, i, 0)),
                  pl.BlockSpec((_H, H2), lambda i: (0, 0)),
                  pl.BlockSpec((1, H2), lambda i: (0, 0)),
                  pl.BlockSpec((H2, H4), lambda i: (0, 0)),
                  pl.BlockSpec((1, H4), lambda i: (0, 0)),
                  pl.BlockSpec((H4, NC), lambda i: (0, 0)),
                  pl.BlockSpec((1, NC), lambda i: (0, 0))],
        out_specs=pl.BlockSpec((_BN, NC), lambda i: (i, 0)),
        out_shape=jax.ShapeDtypeStruct((_N, NC), jnp.float32),
    )(hprev, ahprev, acc, W0, b0, W1, b1, W2, b2)


def kernel(h, edge_index, e, emb_h, We, be, layerW, layerB, W0, b0, W1, b1, W2, b2):
    L = layerW.shape[0]
    src = edge_index[0].astype(jnp.int32)
    dst = edge_index[1].astype(jnp.int32)
    src_g = src.reshape(_NSC, _NS, _NCHG, _RPC, _IR)
    dst_g = dst.reshape(_NSC, _NS, _NCHG, _RPC, _IR)
    src_s = src.reshape(_NS, _NCH, _RPCS, _IR)
    dst_s = dst.reshape(_NS, _NCH, _RPCS, _IR)

    h2 = h.astype(jnp.int32).reshape(_N, 1)
    lW = layerW.astype(jnp.float32)
    lB = layerB.reshape(L, 5, 1, _H).astype(jnp.float32)

    hh = ah = acc = ee = sig = None
    for l in range(L):
        if l == 0:
            hh, ah, bh, dh, eh = _tc_node0(h2, emb_h, lW[0], lB[0])
        else:
            hh, ah, bh, dh, eh = _tc_node(hh, ah, acc, lW[l], lB[l])
        g = _sc_gather(dh, eh, src_g, dst_g)
        w4 = lW[l, 4]
        b4 = lB[l, 4]
        if l == 0:
            sig, ee = _tc_edge0(e, We.reshape(1, _H), be.reshape(1, _H),
                                g, w4, b4)
        elif l < L - 1:
            sig, ee = _tc_edge(ee, g, w4, b4, True)
        else:
            (sig,) = _tc_edge(ee, g, w4, b4, False)
        acc = _sc_scatter(sig, bh, src_s, dst_s)

    return _tc_readout(hh, ah, acc, W0, b0.reshape(1, -1), W1, b1.reshape(1, -1),
                       W2, b2.reshape(1, -1))
